# baseline (speedup 1.0000x reference)
import functools

import jax
import jax.numpy as jnp
from jax import lax
from jax.experimental import pallas as pl
from jax.experimental.pallas import tpu as pltpu
from jax.experimental.pallas import tpu_sc as plsc

N = 10000
E = 320000
DIN = 128
DH = 256
HALF = 128
NSUB = 16
NCORE = 2
NPAD = 10240
ZROWS = NPAD // NSUB
CHUNK = 64

NROW = 5
NIDX = 2 * NROW
EPS = E // NSUB
NCHUNK_A = -(-EPS // (CHUNK * NIDX)) * NIDX
EPS_PAD = NCHUNK_A * CHUNK
NBODY = NCHUNK_A // NIDX

NCHUNK_D = NCHUNK_A // NCORE

RB = 512
LROW = NPAD // 128

_sc_mesh = plsc.VectorSubcoreMesh(core_axis_name="c", subcore_axis_name="s")


DEGW = 128


@functools.partial(
    pl.kernel,
    out_type=(jax.ShapeDtypeStruct((NPAD, DEGW), jnp.float32),
              jax.ShapeDtypeStruct((NPAD, DEGW), jnp.float32)),
    mesh=_sc_mesh,
    scratch_types=[
        pltpu.VMEM((NCHUNK_D, CHUNK), jnp.int32),
        pltpu.VMEM((CHUNK, DEGW), jnp.float32),
        pltpu.VMEM_SHARED((NPAD, DEGW), jnp.float32),
        pltpu.SemaphoreType.DMA,
    ],
)
def _deg_kernel(dstp, zdeg, ones, deg0, deg1, idx_v, ones_v, acc, sem):
    c = lax.axis_index("c")
    s = lax.axis_index("s")
    pltpu.sync_copy(zdeg, acc.at[pl.ds(s * ZROWS, ZROWS)])
    pltpu.sync_copy(ones, ones_v)
    pltpu.sync_copy(dstp.at[s, pl.ds(c * NCHUNK_D, NCHUNK_D)], idx_v)
    plsc.subcore_barrier()

    def body(k, carry):
        pltpu.async_copy(ones_v, acc.at[idx_v.at[k]], sem, add=True)
        return carry

    lax.fori_loop(0, NCHUNK_D, body, 0)

    def drain(k, carry):
        pltpu.make_async_copy(ones_v, acc.at[idx_v.at[0]], sem).wait()
        return carry

    lax.fori_loop(0, NCHUNK_D, drain, 0)
    plsc.subcore_barrier()

    @pl.when(c == 0)
    def _():
        pltpu.sync_copy(acc.at[pl.ds(s * ZROWS, ZROWS)],
                        deg0.at[pl.ds(s * ZROWS, ZROWS)])

    @pl.when(c == 1)
    def _():
        pltpu.sync_copy(acc.at[pl.ds(s * ZROWS, ZROWS)],
                        deg1.at[pl.ds(s * ZROWS, ZROWS)])


@functools.partial(
    pl.kernel,
    out_type=(jax.ShapeDtypeStruct((NPAD, HALF), jnp.float32),
              jax.ShapeDtypeStruct((NPAD, HALF), jnp.float32)),
    mesh=_sc_mesh,
    scratch_types=[
        [pltpu.VMEM((2, CHUNK), jnp.int32) for _ in range(NIDX)],
        [pltpu.VMEM((CHUNK, HALF), jnp.float32) for _ in range(NROW)],
        pltpu.VMEM_SHARED((NPAD, HALF), jnp.float32),
        [pltpu.SemaphoreType.DMA for _ in range(NIDX)],
        [pltpu.SemaphoreType.DMA for _ in range(NROW)],
        [pltpu.SemaphoreType.DMA for _ in range(NROW)],
    ],
)
def _agg_kernel(edges, h0, h1, zmat, out0, out1,
                ibuf, rows, acc, isem, gsem, ssem):
    c = lax.axis_index("c")
    s = lax.axis_index("s")
    pltpu.sync_copy(zmat, acc.at[pl.ds(s * ZROWS, ZROWS)])
    plsc.subcore_barrier()

    def load_idx(k, q):
        return pltpu.async_copy(edges.at[s, k], ibuf[q], isem[q])

    def wait_idx(q):
        pltpu.make_async_copy(edges.at[s, 0], ibuf[q], isem[q]).wait()

    def gather(h, q, p):
        return pltpu.async_copy(h.at[ibuf[q].at[0]], rows[p], gsem[p])

    def wait_gather(h, p):
        pltpu.make_async_copy(h.at[ibuf[0].at[0]], rows[p], gsem[p]).wait()

    def scatter(q, p):
        return pltpu.async_copy(rows[p], acc.at[ibuf[q].at[1]],
                                ssem[p], add=True)

    def run(h):
        for q in range(NIDX):
            load_idx(q, q)
        for p in range(NROW):
            wait_idx(p)
            gather(h, p, p)

        def body(i, carry):
            k = i * NIDX
            for j in range(NIDX):
                q = j
                p = j % NROW
                wait_gather(h, p)
                sd = scatter(q, p)
                sd.wait()
                nxt = k + j + NROW
                qn = (j + NROW) % NIDX

                @pl.when(nxt < NCHUNK_A)
                def _():
                    wait_idx(qn)
                    gather(h, qn, p)

                lda = k + j + NIDX

                @pl.when(lda < NCHUNK_A)
                def _():
                    load_idx(lda, q)
            return carry

        lax.fori_loop(0, NBODY, body, 0)

    @pl.when(c == 0)
    def _():
        run(h0)

    @pl.when(c == 1)
    def _():
        run(h1)

    plsc.subcore_barrier()

    @pl.when(c == 0)
    def _():
        pltpu.sync_copy(acc.at[pl.ds(s * ZROWS, ZROWS)],
                        out0.at[pl.ds(s * ZROWS, ZROWS)])

    @pl.when(c == 1)
    def _():
        pltpu.sync_copy(acc.at[pl.ds(s * ZROWS, ZROWS)],
                        out1.at[pl.ds(s * ZROWS, ZROWS)])


def _dinv_body(deg_ref, out_ref):
    out_ref[...] = lax.rsqrt(deg_ref[...] + 1.0)


_dinv_call = pl.pallas_call(
    _dinv_body,
    out_shape=jax.ShapeDtypeStruct((LROW, 128), jnp.float32),
)


def _mm1_body(x_ref, w_ref, dv_ref, o0_ref, o1_ref):
    h = jnp.dot(x_ref[...], w_ref[...], preferred_element_type=jnp.float32)
    d = dv_ref[...]
    o0_ref[...] = h[:, :HALF] * d
    o1_ref[...] = h[:, HALF:] * d


_mm1_call = pl.pallas_call(
    _mm1_body,
    grid=(NPAD // RB,),
    in_specs=[
        pl.BlockSpec((RB, DIN), lambda i: (i, 0)),
        pl.BlockSpec((DIN, DH), lambda i: (0, 0)),
        pl.BlockSpec((RB, HALF), lambda i: (i, 0)),
    ],
    out_specs=[pl.BlockSpec((RB, HALF), lambda i: (i, 0))] * 2,
    out_shape=[jax.ShapeDtypeStruct((NPAD, HALF), jnp.float32)] * 2,
)


def _mid_body(a0_ref, a1_ref, p0_ref, p1_ref, dv_ref, b_ref, w2_ref,
              o0_ref, o1_ref):
    d = dv_ref[...]
    b = b_ref[...]
    h0 = jnp.maximum(d * (a0_ref[...] + p0_ref[...]) + b[:, :HALF], 0.0)
    h1 = jnp.maximum(d * (a1_ref[...] + p1_ref[...]) + b[:, HALF:], 0.0)
    w2 = w2_ref[...]
    h2 = (jnp.dot(h0, w2[:HALF], preferred_element_type=jnp.float32)
          + jnp.dot(h1, w2[HALF:], preferred_element_type=jnp.float32))
    o0_ref[...] = h2[:, :HALF] * d
    o1_ref[...] = h2[:, HALF:] * d


_mid_call = pl.pallas_call(
    _mid_body,
    grid=(NPAD // RB,),
    in_specs=[
        pl.BlockSpec((RB, HALF), lambda i: (i, 0)),
        pl.BlockSpec((RB, HALF), lambda i: (i, 0)),
        pl.BlockSpec((RB, HALF), lambda i: (i, 0)),
        pl.BlockSpec((RB, HALF), lambda i: (i, 0)),
        pl.BlockSpec((RB, HALF), lambda i: (i, 0)),
        pl.BlockSpec((1, DH), lambda i: (0, 0)),
        pl.BlockSpec((DH, DH), lambda i: (0, 0)),
    ],
    out_specs=[pl.BlockSpec((RB, HALF), lambda i: (i, 0))] * 2,
    out_shape=[jax.ShapeDtypeStruct((NPAD, HALF), jnp.float32)] * 2,
)


def _fin_body(a0_ref, a1_ref, p0_ref, p1_ref, dv_ref, b_ref, o_ref):
    d = dv_ref[...]
    b = b_ref[...]
    r0 = d * (a0_ref[...] + p0_ref[...]) + b[:, :HALF]
    r1 = d * (a1_ref[...] + p1_ref[...]) + b[:, HALF:]
    o_ref[...] = jnp.concatenate([r0, r1], axis=1)


_fin_call = pl.pallas_call(
    _fin_body,
    grid=(NPAD // RB,),
    in_specs=[
        pl.BlockSpec((RB, HALF), lambda i: (i, 0)),
        pl.BlockSpec((RB, HALF), lambda i: (i, 0)),
        pl.BlockSpec((RB, HALF), lambda i: (i, 0)),
        pl.BlockSpec((RB, HALF), lambda i: (i, 0)),
        pl.BlockSpec((RB, HALF), lambda i: (i, 0)),
        pl.BlockSpec((1, DH), lambda i: (0, 0)),
    ],
    out_specs=pl.BlockSpec((RB, DH), lambda i: (i, 0)),
    out_shape=jax.ShapeDtypeStruct((NPAD, DH), jnp.float32),
)


def kernel(x, edge_index, W1, b1, W2, b2):
    src = edge_index[0].astype(jnp.int32)
    dst = edge_index[1].astype(jnp.int32)

    srcp = jnp.concatenate(
        [src.reshape(NSUB, EPS),
         jnp.zeros((NSUB, EPS_PAD - EPS), jnp.int32)],
        axis=1).reshape(NSUB, NCHUNK_A, CHUNK)
    dstp = jnp.concatenate(
        [dst.reshape(NSUB, EPS),
         jnp.full((NSUB, EPS_PAD - EPS), N, jnp.int32)],
        axis=1).reshape(NSUB, NCHUNK_A, CHUNK)
    edges = jnp.stack([srcp, dstp], axis=2)

    zdeg = jnp.zeros((ZROWS, DEGW), jnp.float32)
    zmat = jnp.zeros((ZROWS, HALF), jnp.float32)
    ones = jnp.ones((CHUNK, DEGW), jnp.float32)

    deg0, deg1 = _deg_kernel(dstp, zdeg, ones)
    dsum = deg0[:, 0] + deg1[:, 0]
    dinv = _dinv_call(dsum.reshape(LROW, 128))
    dinvb = jnp.broadcast_to(dinv.reshape(NPAD, 1), (NPAD, HALF))

    xp = jnp.concatenate([x, jnp.zeros((NPAD - N, DIN), x.dtype)], axis=0)
    hp0, hp1 = _mm1_call(xp, W1, dinvb)
    a0, a1 = _agg_kernel(edges, hp0, hp1, zmat)
    q0, q1 = _mid_call(a0, a1, hp0, hp1, dinvb, b1.reshape(1, DH), W2)
    a0, a1 = _agg_kernel(edges, q0, q1, zmat)
    emb = _fin_call(a0, a1, q0, q1, dinvb, b2.reshape(1, DH))
    return emb[:N]


# speedup vs baseline: 1.0642x; 1.0642x over previous
import functools

import jax
import jax.numpy as jnp
from jax import lax
from jax.experimental import pallas as pl
from jax.experimental.pallas import tpu as pltpu
from jax.experimental.pallas import tpu_sc as plsc

N = 10000
E = 320000
DIN = 128
DH = 256
HALF = 128
NSUB = 16
NCORE = 2
NPAD = 10240
ZROWS = NPAD // NSUB
CHUNK = 64

NROW = 5
NIDX = 2 * NROW
EPS = E // NSUB
NCHUNK_A = -(-EPS // (CHUNK * NIDX)) * NIDX
EPS_PAD = NCHUNK_A * CHUNK
NBODY = NCHUNK_A // NIDX

NCHUNK_D = NCHUNK_A // NCORE

RB = 512
LROW = NPAD // 128

_sc_mesh = plsc.VectorSubcoreMesh(core_axis_name="c", subcore_axis_name="s")


@functools.partial(
    pl.kernel,
    out_type=(jax.ShapeDtypeStruct((NPAD,), jnp.float32),
              jax.ShapeDtypeStruct((NPAD,), jnp.float32)),
    mesh=_sc_mesh,
    scratch_types=[
        pltpu.VMEM((NCHUNK_D, CHUNK), jnp.int32),
        pltpu.VMEM((CHUNK,), jnp.float32),
        pltpu.VMEM_SHARED((NPAD,), jnp.float32),
        pltpu.SemaphoreType.DMA,
    ],
)
def _deg_kernel(dstp, zvec, ones, deg0, deg1, idx_v, ones_v, acc, sem):
    c = lax.axis_index("c")
    s = lax.axis_index("s")
    pltpu.sync_copy(zvec, acc.at[pl.ds(s * ZROWS, ZROWS)])
    pltpu.sync_copy(ones, ones_v)
    pltpu.sync_copy(dstp.at[s, pl.ds(c * NCHUNK_D, NCHUNK_D)], idx_v)
    plsc.subcore_barrier()

    def body(k, carry):
        pltpu.async_copy(ones_v, acc.at[idx_v.at[k]], sem, add=True)
        return carry

    lax.fori_loop(0, NCHUNK_D, body, 0)

    def drain(k, carry):
        pltpu.make_async_copy(ones_v, acc.at[idx_v.at[0]], sem).wait()
        return carry

    lax.fori_loop(0, NCHUNK_D, drain, 0)
    plsc.subcore_barrier()

    @pl.when(c == 0)
    def _():
        pltpu.sync_copy(acc.at[pl.ds(s * ZROWS, ZROWS)],
                        deg0.at[pl.ds(s * ZROWS, ZROWS)])

    @pl.when(c == 1)
    def _():
        pltpu.sync_copy(acc.at[pl.ds(s * ZROWS, ZROWS)],
                        deg1.at[pl.ds(s * ZROWS, ZROWS)])


@functools.partial(
    pl.kernel,
    out_type=(jax.ShapeDtypeStruct((NPAD, HALF), jnp.float32),
              jax.ShapeDtypeStruct((NPAD, HALF), jnp.float32)),
    mesh=_sc_mesh,
    scratch_types=[
        [pltpu.VMEM((2, CHUNK), jnp.int32) for _ in range(NIDX)],
        [pltpu.VMEM((CHUNK, HALF), jnp.float32) for _ in range(NROW)],
        pltpu.VMEM_SHARED((NPAD, HALF), jnp.float32),
        [pltpu.SemaphoreType.DMA for _ in range(NIDX)],
        [pltpu.SemaphoreType.DMA for _ in range(NROW)],
        [pltpu.SemaphoreType.DMA for _ in range(NROW)],
    ],
)
def _agg_kernel(edges, h0, h1, zmat, out0, out1,
                ibuf, rows, acc, isem, gsem, ssem):
    c = lax.axis_index("c")
    s = lax.axis_index("s")
    pltpu.sync_copy(zmat, acc.at[pl.ds(s * ZROWS, ZROWS)])
    plsc.subcore_barrier()

    def load_idx(k, q):
        return pltpu.async_copy(edges.at[s, k], ibuf[q], isem[q])

    def wait_idx(q):
        pltpu.make_async_copy(edges.at[s, 0], ibuf[q], isem[q]).wait()

    def gather(h, q, p):
        return pltpu.async_copy(h.at[ibuf[q].at[0]], rows[p], gsem[p])

    def wait_gather(h, p):
        pltpu.make_async_copy(h.at[ibuf[0].at[0]], rows[p], gsem[p]).wait()

    def scatter(q, p):
        return pltpu.async_copy(rows[p], acc.at[ibuf[q].at[1]],
                                ssem[p], add=True)

    def run(h):
        for q in range(NIDX):
            load_idx(q, q)
        for p in range(NROW):
            wait_idx(p)
            gather(h, p, p)

        def body(i, carry):
            k = i * NIDX
            for j in range(NIDX):
                q = j
                p = j % NROW
                wait_gather(h, p)
                sd = scatter(q, p)
                sd.wait()
                nxt = k + j + NROW
                qn = (j + NROW) % NIDX

                @pl.when(nxt < NCHUNK_A)
                def _():
                    wait_idx(qn)
                    gather(h, qn, p)

                lda = k + j + NIDX

                @pl.when(lda < NCHUNK_A)
                def _():
                    load_idx(lda, q)
            return carry

        lax.fori_loop(0, NBODY, body, 0)

    @pl.when(c == 0)
    def _():
        run(h0)

    @pl.when(c == 1)
    def _():
        run(h1)

    plsc.subcore_barrier()

    @pl.when(c == 0)
    def _():
        pltpu.sync_copy(acc.at[pl.ds(s * ZROWS, ZROWS)],
                        out0.at[pl.ds(s * ZROWS, ZROWS)])

    @pl.when(c == 1)
    def _():
        pltpu.sync_copy(acc.at[pl.ds(s * ZROWS, ZROWS)],
                        out1.at[pl.ds(s * ZROWS, ZROWS)])


def _dinv_body(deg_ref, out_ref):
    out_ref[...] = lax.rsqrt(deg_ref[...] + 1.0)


_dinv_call = pl.pallas_call(
    _dinv_body,
    out_shape=jax.ShapeDtypeStruct((LROW, 128), jnp.float32),
)


def _mm1_body(x_ref, w_ref, dv_ref, o0_ref, o1_ref):
    h = jnp.dot(x_ref[...], w_ref[...], preferred_element_type=jnp.float32)
    d = dv_ref[...]
    o0_ref[...] = h[:, :HALF] * d
    o1_ref[...] = h[:, HALF:] * d


_mm1_call = pl.pallas_call(
    _mm1_body,
    grid=(NPAD // RB,),
    in_specs=[
        pl.BlockSpec((RB, DIN), lambda i: (i, 0)),
        pl.BlockSpec((DIN, DH), lambda i: (0, 0)),
        pl.BlockSpec((RB, HALF), lambda i: (i, 0)),
    ],
    out_specs=[pl.BlockSpec((RB, HALF), lambda i: (i, 0))] * 2,
    out_shape=[jax.ShapeDtypeStruct((NPAD, HALF), jnp.float32)] * 2,
)


def _mid_body(a0_ref, a1_ref, p0_ref, p1_ref, dv_ref, b_ref, w2_ref,
              o0_ref, o1_ref):
    d = dv_ref[...]
    b = b_ref[...]
    h0 = jnp.maximum(d * (a0_ref[...] + p0_ref[...]) + b[:, :HALF], 0.0)
    h1 = jnp.maximum(d * (a1_ref[...] + p1_ref[...]) + b[:, HALF:], 0.0)
    w2 = w2_ref[...]
    h2 = (jnp.dot(h0, w2[:HALF], preferred_element_type=jnp.float32)
          + jnp.dot(h1, w2[HALF:], preferred_element_type=jnp.float32))
    o0_ref[...] = h2[:, :HALF] * d
    o1_ref[...] = h2[:, HALF:] * d


_mid_call = pl.pallas_call(
    _mid_body,
    grid=(NPAD // RB,),
    in_specs=[
        pl.BlockSpec((RB, HALF), lambda i: (i, 0)),
        pl.BlockSpec((RB, HALF), lambda i: (i, 0)),
        pl.BlockSpec((RB, HALF), lambda i: (i, 0)),
        pl.BlockSpec((RB, HALF), lambda i: (i, 0)),
        pl.BlockSpec((RB, HALF), lambda i: (i, 0)),
        pl.BlockSpec((1, DH), lambda i: (0, 0)),
        pl.BlockSpec((DH, DH), lambda i: (0, 0)),
    ],
    out_specs=[pl.BlockSpec((RB, HALF), lambda i: (i, 0))] * 2,
    out_shape=[jax.ShapeDtypeStruct((NPAD, HALF), jnp.float32)] * 2,
)


def _fin_body(a0_ref, a1_ref, p0_ref, p1_ref, dv_ref, b_ref, o_ref):
    d = dv_ref[...]
    b = b_ref[...]
    r0 = d * (a0_ref[...] + p0_ref[...]) + b[:, :HALF]
    r1 = d * (a1_ref[...] + p1_ref[...]) + b[:, HALF:]
    o_ref[...] = jnp.concatenate([r0, r1], axis=1)


_fin_call = pl.pallas_call(
    _fin_body,
    grid=(NPAD // RB,),
    in_specs=[
        pl.BlockSpec((RB, HALF), lambda i: (i, 0)),
        pl.BlockSpec((RB, HALF), lambda i: (i, 0)),
        pl.BlockSpec((RB, HALF), lambda i: (i, 0)),
        pl.BlockSpec((RB, HALF), lambda i: (i, 0)),
        pl.BlockSpec((RB, HALF), lambda i: (i, 0)),
        pl.BlockSpec((1, DH), lambda i: (0, 0)),
    ],
    out_specs=pl.BlockSpec((RB, DH), lambda i: (i, 0)),
    out_shape=jax.ShapeDtypeStruct((NPAD, DH), jnp.float32),
)


def kernel(x, edge_index, W1, b1, W2, b2):
    src = edge_index[0].astype(jnp.int32)
    dst = edge_index[1].astype(jnp.int32)

    srcp = jnp.concatenate(
        [src.reshape(NSUB, EPS),
         jnp.zeros((NSUB, EPS_PAD - EPS), jnp.int32)],
        axis=1).reshape(NSUB, NCHUNK_A, CHUNK)
    dstp = jnp.concatenate(
        [dst.reshape(NSUB, EPS),
         jnp.full((NSUB, EPS_PAD - EPS), N, jnp.int32)],
        axis=1).reshape(NSUB, NCHUNK_A, CHUNK)
    edges = jnp.stack([srcp, dstp], axis=2)

    zvec = jnp.zeros((ZROWS,), jnp.float32)
    zmat = jnp.zeros((ZROWS, HALF), jnp.float32)
    ones = jnp.ones((CHUNK,), jnp.float32)

    deg0, deg1 = _deg_kernel(dstp, zvec, ones)
    dinv = _dinv_call((deg0 + deg1).reshape(LROW, 128))
    dinvb = jnp.broadcast_to(dinv.reshape(NPAD, 1), (NPAD, HALF))

    xp = jnp.concatenate([x, jnp.zeros((NPAD - N, DIN), x.dtype)], axis=0)
    hp0, hp1 = _mm1_call(xp, W1, dinvb)
    a0, a1 = _agg_kernel(edges, hp0, hp1, zmat)
    q0, q1 = _mid_call(a0, a1, hp0, hp1, dinvb, b1.reshape(1, DH), W2)
    a0, a1 = _agg_kernel(edges, q0, q1, zmat)
    emb = _fin_call(a0, a1, q0, q1, dinvb, b2.reshape(1, DH))
    return emb[:N]
